# baseline (speedup 1.0000x reference)
import functools

import jax
import jax.numpy as jnp
from jax import lax
from jax.experimental import pallas as pl
from jax.experimental.pallas import tpu as pltpu
from jax.experimental.pallas import tpu_sc as plsc

_HALF = 64
_SEQ = 8192
_NC = 2
_NS = 16
_NW = _NC * _NS
_BPW = _SEQ // _NW
_L = 16
_NCHUNK = 4
_ROWS_PER_CHUNK = _BPW // _NCHUNK

_INV_2PI = 0.15915494309189535
_TWO_PI_HI = 6.2831854820251465
_TWO_PI_LO = -1.7484556000744883e-07
_PI = 3.14159265358979

_COS_C = (-0.9999994435770305, 0.49999558143188294, -0.04166103265415857,
          0.001386274698146315, -2.425318891836198e-05,
          2.2193936088932276e-07)
_SIN_C = (0.9999997069588598, -0.1666657719811158, 0.008332557998428487,
          -0.00019812572237797466, 2.704047331408832e-06,
          -2.0534080102940777e-08)


def _poly(coeffs, t):
    acc = jnp.full((_L,), coeffs[-1], dtype=jnp.float32)
    for c in coeffs[-2::-1]:
        acc = acc * t + jnp.float32(c)
    return acc


_sc_mesh = plsc.VectorSubcoreMesh(
    core_axis_name="c", subcore_axis_name="s",
    num_cores=_NC, num_subcores=_NS,
)


@functools.partial(
    pl.kernel,
    mesh=_sc_mesh,
    out_type=jax.ShapeDtypeStruct((_SEQ, _HALF), jnp.float32),
    scratch_types=[
        pltpu.VMEM((_BPW,), jnp.int32),
        pltpu.VMEM((_HALF,), jnp.float32),
        pltpu.VMEM((_BPW, _HALF), jnp.float32),
        pltpu.SemaphoreType.DMA,
    ],
    compiler_params=pltpu.CompilerParams(use_tc_tiling_on_sc=False),
)
def _sc_rotary(pos_hbm, invf_hbm, cos_out,
               idx_v, invf_v, cos_v, sem):
    wid = lax.axis_index("s") * _NC + lax.axis_index("c")
    base = wid * _BPW
    pltpu.sync_copy(pos_hbm.at[pl.ds(base, _BPW)], idx_v)
    pltpu.sync_copy(invf_hbm, invf_v)

    freqs = [invf_v[pl.ds(k * _L, _L)] * jnp.float32(_INV_2PI)
             for k in range(_HALF // _L)]

    @plsc.parallel_loop(0, _BPW // _L, unroll=4)
    def _loop(i):
        pv = idx_v[pl.ds(i * _L, _L)].astype(jnp.float32)
        for j in range(_L):
            row = i * _L + j
            pf = jnp.full((_L,), pv[j], jnp.float32)
            for k, fv in enumerate(freqs):
                w = pf * fv
                frac = w - w.astype(jnp.int32).astype(jnp.float32)
                v = jnp.float32(_PI) - frac * jnp.float32(_TWO_PI_HI)
                t = v * v
                cos_v[row, pl.ds(k * _L, _L)] = _poly(_COS_C, t)

    pltpu.async_copy(cos_v, cos_out.at[pl.ds(base, _BPW)], sem).wait()


_TCBLK = 2048


def _tc_sin_body(pos_ref, invf_ref, sin_ref):
    pos = pos_ref[...].astype(jnp.float32)
    x = pos * invf_ref[...]
    n = (x * jnp.float32(_INV_2PI)).astype(jnp.int32)
    nf = n.astype(jnp.float32)
    v = jnp.float32(_PI) - x
    v = v + nf * jnp.float32(_TWO_PI_HI)
    v = v + nf * jnp.float32(_TWO_PI_LO)
    t = v * v
    acc = jnp.full(x.shape, _SIN_C[-1], dtype=jnp.float32)
    for c in _SIN_C[-2::-1]:
        acc = acc * t + jnp.float32(c)
    sin_ref[...] = v * acc


_tc_sin = pl.pallas_call(
    _tc_sin_body,
    grid=(_SEQ // _TCBLK,),
    in_specs=[
        pl.BlockSpec((_TCBLK, 1), lambda i: (i, 0)),
        pl.BlockSpec((1, _HALF), lambda i: (0, 0)),
    ],
    out_specs=pl.BlockSpec((_TCBLK, _HALF), lambda i: (i, 0)),
    out_shape=jax.ShapeDtypeStruct((_SEQ, _HALF), jnp.float32),
)


def kernel(positions, inv_freq):
    pos_i32 = positions.astype(jnp.int32)
    cos = _sc_rotary(pos_i32, inv_freq)
    sin = _tc_sin(pos_i32.reshape(_SEQ, 1), inv_freq.reshape(1, _HALF))
    return (cos, sin)


# speedup vs baseline: 1.1743x; 1.1743x over previous
import functools

import jax
import jax.numpy as jnp
from jax import lax
from jax.experimental import pallas as pl
from jax.experimental.pallas import tpu as pltpu
from jax.experimental.pallas import tpu_sc as plsc

_HALF = 64
_SEQ = 8192
_NC = 2
_NS = 16
_NW = _NC * _NS
_BPW = _SEQ // _NW
_L = 16
_NCHUNK = 4
_ROWS_PER_CHUNK = _BPW // _NCHUNK

_INV_2PI = 0.15915494309189535
_TWO_PI_HI = 6.2831854820251465
_TWO_PI_LO = -1.7484556000744883e-07
_PI = 3.14159265358979

_COS_C = (-0.9999994435770305, 0.49999558143188294, -0.04166103265415857,
          0.001386274698146315, -2.425318891836198e-05,
          2.2193936088932276e-07)
_SIN_C = (0.9999997069588598, -0.1666657719811158, 0.008332557998428487,
          -0.00019812572237797466, 2.704047331408832e-06,
          -2.0534080102940777e-08)


def _poly(coeffs, t):
    acc = jnp.full((_L,), coeffs[-1], dtype=jnp.float32)
    for c in coeffs[-2::-1]:
        acc = acc * t + jnp.float32(c)
    return acc


_sc_mesh = plsc.VectorSubcoreMesh(
    core_axis_name="c", subcore_axis_name="s",
    num_cores=_NC, num_subcores=_NS,
)


@functools.partial(
    pl.kernel,
    mesh=_sc_mesh,
    out_type=jax.ShapeDtypeStruct((_SEQ, _HALF), jnp.float32),
    scratch_types=[
        pltpu.VMEM((_BPW,), jnp.int32),
        pltpu.VMEM((_HALF,), jnp.float32),
        pltpu.VMEM((_BPW, _HALF), jnp.float32),
        pltpu.SemaphoreType.DMA,
    ],
    compiler_params=pltpu.CompilerParams(use_tc_tiling_on_sc=False),
)
def _sc_rotary(pos_hbm, invf_hbm, cos_out,
               idx_v, invf_v, cos_v, sem):
    wid = lax.axis_index("s") * _NC + lax.axis_index("c")
    base = wid * _BPW
    pltpu.sync_copy(pos_hbm.at[pl.ds(base, _BPW)], idx_v)
    pltpu.sync_copy(invf_hbm, invf_v)

    freqs = [invf_v[pl.ds(k * _L, _L)] * jnp.float32(_INV_2PI)
             for k in range(_HALF // _L)]

    @plsc.parallel_loop(0, _BPW // _L, unroll=2)
    def _loop(i):
        pv = idx_v[pl.ds(i * _L, _L)].astype(jnp.float32)
        for j in range(_L):
            row = i * _L + j
            pf = jnp.full((_L,), pv[j], jnp.float32)
            for k, fv in enumerate(freqs):
                w = pf * fv
                frac = w - w.astype(jnp.int32).astype(jnp.float32)
                v = jnp.float32(_PI) - frac * jnp.float32(_TWO_PI_HI)
                t = v * v
                cos_v[row, pl.ds(k * _L, _L)] = _poly(_COS_C, t)

    pltpu.async_copy(cos_v, cos_out.at[pl.ds(base, _BPW)], sem).wait()


_TCBLK = 2048


def _tc_sin_body(pos_ref, invf_ref, sin_ref):
    pos = pos_ref[...].astype(jnp.float32)
    x = pos * invf_ref[...]
    n = (x * jnp.float32(_INV_2PI)).astype(jnp.int32)
    nf = n.astype(jnp.float32)
    v = jnp.float32(_PI) - x
    v = v + nf * jnp.float32(_TWO_PI_HI)
    v = v + nf * jnp.float32(_TWO_PI_LO)
    t = v * v
    acc = jnp.full(x.shape, _SIN_C[-1], dtype=jnp.float32)
    for c in _SIN_C[-2::-1]:
        acc = acc * t + jnp.float32(c)
    sin_ref[...] = v * acc


_tc_sin = pl.pallas_call(
    _tc_sin_body,
    grid=(_SEQ // _TCBLK,),
    in_specs=[
        pl.BlockSpec((_TCBLK, 1), lambda i: (i, 0)),
        pl.BlockSpec((1, _HALF), lambda i: (0, 0)),
    ],
    out_specs=pl.BlockSpec((_TCBLK, _HALF), lambda i: (i, 0)),
    out_shape=jax.ShapeDtypeStruct((_SEQ, _HALF), jnp.float32),
)


def kernel(positions, inv_freq):
    pos_i32 = positions.astype(jnp.int32)
    cos = _sc_rotary(pos_i32, inv_freq)
    sin = _tc_sin(pos_i32.reshape(_SEQ, 1), inv_freq.reshape(1, _HALF))
    return (cos, sin)


# speedup vs baseline: 1.1750x; 1.0006x over previous
import functools

import jax
import jax.numpy as jnp
from jax import lax
from jax.experimental import pallas as pl
from jax.experimental.pallas import tpu as pltpu
from jax.experimental.pallas import tpu_sc as plsc

_HALF = 64
_SEQ = 8192
_NC = 2
_NS = 16
_NW = _NC * _NS
_BPW = _SEQ // _NW
_L = 16
_INV_2PI = 0.15915494309189535
_TWO_PI_HI = 6.2831854820251465
_TWO_PI_LO = -1.7484556000744883e-07
_PI = 3.14159265358979

_COS_C = (-0.9999994435770305, 0.49999558143188294, -0.04166103265415857,
          0.001386274698146315, -2.425318891836198e-05,
          2.2193936088932276e-07)
_SIN_C = (0.9999997069588598, -0.1666657719811158, 0.008332557998428487,
          -0.00019812572237797466, 2.704047331408832e-06,
          -2.0534080102940777e-08)


def _poly(coeffs, t):
    acc = jnp.full((_L,), coeffs[-1], dtype=jnp.float32)
    for c in coeffs[-2::-1]:
        acc = acc * t + jnp.float32(c)
    return acc


_sc_mesh = plsc.VectorSubcoreMesh(
    core_axis_name="c", subcore_axis_name="s",
    num_cores=_NC, num_subcores=_NS,
)


@functools.partial(
    pl.kernel,
    mesh=_sc_mesh,
    out_type=jax.ShapeDtypeStruct((_SEQ, _HALF), jnp.float32),
    scratch_types=[
        pltpu.VMEM((_BPW,), jnp.int32),
        pltpu.VMEM((_HALF,), jnp.float32),
        pltpu.VMEM((_BPW, _HALF), jnp.float32),
        pltpu.SemaphoreType.DMA,
    ],
    compiler_params=pltpu.CompilerParams(use_tc_tiling_on_sc=False),
)
def _sc_rotary(pos_hbm, invf_hbm, cos_out,
               idx_v, invf_v, cos_v, sem):
    wid = lax.axis_index("s") * _NC + lax.axis_index("c")
    base = wid * _BPW
    pltpu.sync_copy(pos_hbm.at[pl.ds(base, _BPW)], idx_v)
    pltpu.sync_copy(invf_hbm, invf_v)

    freqs = [invf_v[pl.ds(k * _L, _L)] * jnp.float32(_INV_2PI)
             for k in range(_HALF // _L)]

    @plsc.parallel_loop(0, _BPW // _L, unroll=2)
    def _loop(i):
        pv = idx_v[pl.ds(i * _L, _L)].astype(jnp.float32)
        for j in range(_L):
            row = i * _L + j
            pf = jnp.full((_L,), pv[j], jnp.float32)
            for k, fv in enumerate(freqs):
                w = pf * fv
                frac = w - w.astype(jnp.int32).astype(jnp.float32)
                v = jnp.float32(_PI) - frac * jnp.float32(_TWO_PI_HI)
                t = v * v
                cos_v[row, pl.ds(k * _L, _L)] = _poly(_COS_C, t)

    pltpu.async_copy(cos_v, cos_out.at[pl.ds(base, _BPW)], sem).wait()


_TCBLK = 2048


def _tc_sin_body(pos_ref, invf_ref, sin_ref):
    pos = pos_ref[...].astype(jnp.float32)
    x = pos * invf_ref[...]
    n = (x * jnp.float32(_INV_2PI)).astype(jnp.int32)
    nf = n.astype(jnp.float32)
    v = jnp.float32(_PI) - x
    v = v + nf * jnp.float32(_TWO_PI_HI)
    v = v + nf * jnp.float32(_TWO_PI_LO)
    t = v * v
    acc = jnp.full(x.shape, _SIN_C[-1], dtype=jnp.float32)
    for c in _SIN_C[-2::-1]:
        acc = acc * t + jnp.float32(c)
    sin_ref[...] = v * acc


_tc_sin = pl.pallas_call(
    _tc_sin_body,
    grid=(_SEQ // _TCBLK,),
    in_specs=[
        pl.BlockSpec((_TCBLK, 1), lambda i: (i, 0)),
        pl.BlockSpec((1, _HALF), lambda i: (0, 0)),
    ],
    out_specs=pl.BlockSpec((_TCBLK, _HALF), lambda i: (i, 0)),
    out_shape=jax.ShapeDtypeStruct((_SEQ, _HALF), jnp.float32),
)


def kernel(positions, inv_freq):
    pos_i32 = positions.astype(jnp.int32)
    cos = _sc_rotary(pos_i32, inv_freq)
    sin = _tc_sin(pos_i32.reshape(_SEQ, 1), inv_freq.reshape(1, _HALF))
    return (cos, sin)
